# baseline (speedup 1.0000x reference)
import functools

import jax
import jax.numpy as jnp
from jax import lax
from jax.experimental import pallas as pl
from jax.experimental.pallas import tpu as pltpu
from jax.experimental.pallas import tpu_sc as plsc

NV = 100000
E = 3200000
H = 16

NC = 2
NS = 16
NW = NC * NS
EPW = E // NW
CH = 200
NCH = EPW // CH
NRCH = NV // CH
NB = 4
EPT = E // NS
CHD = 2000
NCHD = EPT // CHD
NZCHD = NV // CHD

_mesh = plsc.VectorSubcoreMesh(core_axis_name="c", subcore_axis_name="s")


def _fill(ref, n, value):
    vec = jnp.full((16,), value, ref.dtype)

    @plsc.parallel_loop(0, n, 16)
    def _(i):
        ref[pl.ds(i, 16)] = vec


def _rsqrt16(x):
    i = lax.bitcast_convert_type(x, jnp.int32)
    i = 0x5F3759DF - lax.shift_right_logical(i, 1)
    y = lax.bitcast_convert_type(i, jnp.float32)
    for _ in range(3):
        y = y * (1.5 - 0.5 * x * y * y)
    return y


@functools.partial(
    pl.kernel,
    out_type=(
        jax.ShapeDtypeStruct((12800 * 128,), jnp.float32),
        jax.ShapeDtypeStruct((12800 * 128,), jnp.float32),
    ),
    mesh=_mesh,
    scratch_types=[
        pltpu.VMEM((CHD,), jnp.int32),
        pltpu.VMEM((CHD,), jnp.int32),
        pltpu.VMEM((CHD,), jnp.float32),
        pltpu.VMEM((CHD * H,), jnp.float32),
        pltpu.VMEM_SHARED((NV,), jnp.float32),
        pltpu.SemaphoreType.DMA,
        pltpu.SemaphoreType.DMA,
        pltpu.SemaphoreType.DMA,
        pltpu.SemaphoreType.DMA,
    ],
    compiler_params=pltpu.CompilerParams(use_tc_tiling_on_sc=False),
)
def _degrees(ei_hbm, rv_out, rc_out,
             sv0, sv1, ones_v, stage1d, hist_sh,
             semi0, semi1, sems0, sems1):
    cid = lax.axis_index("c")
    sid = lax.axis_index("s")
    sv = (sv0, sv1)
    semi = (semi0, semi1)
    sems = (sems0, sems1)

    _fill(ones_v, CHD, 0.0)
    for m in range(-(-NZCHD // NS)):
        jj = sid + NS * m

        @pl.when(jj < NZCHD)
        def _():
            pltpu.sync_copy(ones_v, hist_sh.at[pl.ds(jj * CHD, CHD)])
    _fill(ones_v, CHD, 1.0)
    plsc.subcore_barrier()

    def start_idx(j, b):
        base = sid * EPT + j * CHD
        pltpu.async_copy(ei_hbm.at[cid, pl.ds(base, CHD)], sv[b], semi[b])

    def wait_idx(b):
        pltpu.make_async_copy(ei_hbm.at[0, pl.ds(0, CHD)], sv[b], semi[b]).wait()

    def start_scat(b):
        pltpu.async_copy(ones_v, hist_sh.at[sv[b]], sems[b], add=True)

    def wait_scat(b):
        pltpu.make_async_copy(ones_v, hist_sh.at[sv[b]], sems[b]).wait()

    start_idx(0, 0)
    wait_idx(0)
    start_scat(0)
    start_idx(1, 1)

    def pair(t, _):
        for k in range(2):
            j = 2 * t + 1 + k
            b = 1 - k
            wait_idx(b)
            start_scat(b)
            wait_scat(1 - b)

            @pl.when(j < NCHD - 1)
            def _():
                start_idx(j + 1, 1 - b)
        return 0

    lax.fori_loop(0, (NCHD - 1) // 2, pair, 0)
    wait_idx(1)
    start_scat(1)
    wait_scat(0)
    wait_scat(1)
    plsc.subcore_barrier()

    for m in range(-(-NZCHD // NS)):
        jj = sid + NS * m

        @pl.when(jj < NZCHD)
        def _():
            pltpu.sync_copy(hist_sh.at[pl.ds(jj * CHD, CHD)], ones_v)

            @plsc.parallel_loop(0, CHD, 16)
            def _(i):
                d = jnp.maximum(ones_v[pl.ds(i, 16)], 1.0)
                y16 = _rsqrt16(d)
                for k in range(16):
                    stage1d[pl.ds((i + k) * H, H)] = jnp.full((H,), y16[k])

            @pl.when(cid == 0)
            def _():
                pltpu.sync_copy(stage1d, rv_out.at[pl.ds(jj * CHD * H, CHD * H)])

            @pl.when(cid == 1)
            def _():
                pltpu.sync_copy(stage1d, rc_out.at[pl.ds(jj * CHD * H, CHD * H)])


WBR = 800
NWB = NV // WBR


def _make_edge_pass(grow, srow):
    @functools.partial(
        pl.kernel,
        out_type=jax.ShapeDtypeStruct((NC * 12800 * 8, H), jnp.float32),
        mesh=_mesh,
        scratch_types=(
            [pltpu.VMEM((CH,), jnp.int32) for _ in range(NB)]
            + [pltpu.VMEM((CH,), jnp.int32) for _ in range(NB)]
            + [pltpu.VMEM((CH,), jnp.float32) for _ in range(NB)]
            + [pltpu.VMEM((CH, H), jnp.float32) for _ in range(NB)]
            + [pltpu.VMEM((WBR, H), jnp.float32)]
            + [pltpu.VMEM_SHARED((NV, H), jnp.float32)]
            + [pltpu.SemaphoreType.DMA for _ in range(3 * NB)]
        ),
        compiler_params=pltpu.CompilerParams(use_tc_tiling_on_sc=False),
    )
    def _edge_pass(table_hbm, ei_hbm, ew_hbm, out_hbm, *bufs):
        gi = bufs[0:NB]
        si = bufs[NB:2 * NB]
        ew = bufs[2 * NB:3 * NB]
        rows = bufs[3 * NB:4 * NB]
        wb = bufs[4 * NB]
        acc_sh = bufs[4 * NB + 1]
        semi = bufs[4 * NB + 2:4 * NB + 2 + NB]
        semg = bufs[4 * NB + 2 + NB:4 * NB + 2 + 2 * NB]
        sems = bufs[4 * NB + 2 + 2 * NB:4 * NB + 2 + 3 * NB]
        _edge_pass_body(grow, srow, table_hbm, ei_hbm, ew_hbm, out_hbm,
                        gi, si, ew, rows, wb, acc_sh, semi, semg, sems)

    return _edge_pass


def _edge_pass_body(grow, srow, table_hbm, ei_hbm, ew_hbm, out_hbm,
                    gi, si, ew, rows, wb, acc_sh, semi, semg, sems):
    cid = lax.axis_index("c")
    sid = lax.axis_index("s")
    wid = cid * NS + sid

    zvec = jnp.zeros((16,), jnp.float32)

    @plsc.parallel_loop(0, WBR, 1)
    def _(e):
        wb[e, :] = zvec

    for m in range(-(-NWB // NS)):
        jj = sid + NS * m

        @pl.when(jj < NWB)
        def _():
            pltpu.sync_copy(wb, acc_sh.at[pl.ds(jj * WBR, WBR)])
    plsc.subcore_barrier()

    def start_idx(j, b):
        base = wid * EPW + j * CH
        pltpu.async_copy(ei_hbm.at[grow, pl.ds(base, CH)], gi[b], semi[b])
        pltpu.async_copy(ei_hbm.at[srow, pl.ds(base, CH)], si[b], semi[b])
        pltpu.async_copy(ew_hbm.at[pl.ds(base, CH)], ew[b], semi[b])

    def wait_idx(b):
        pltpu.make_async_copy(ei_hbm.at[grow, pl.ds(0, CH)], gi[b], semi[b]).wait()
        pltpu.make_async_copy(ei_hbm.at[srow, pl.ds(0, CH)], si[b], semi[b]).wait()
        pltpu.make_async_copy(ew_hbm.at[pl.ds(0, CH)], ew[b], semi[b]).wait()

    def start_gather(b):
        pltpu.async_copy(table_hbm.at[gi[b]], rows[b], semg[b])

    def wait_gather(b):
        pltpu.make_async_copy(table_hbm.at[gi[b]], rows[b], semg[b]).wait()

    def scale(b):
        rv = rows[b]
        ev = ew[b]

        @plsc.parallel_loop(0, CH, 16)
        def _(e):
            w16 = ev[pl.ds(e, 16)]
            for k in range(16):
                rv[e + k, :] = rv[e + k, :] * w16[k]

    def start_scat(b):
        pltpu.async_copy(rows[b], acc_sh.at[si[b]], sems[b], add=True)

    def wait_scat(b):
        pltpu.make_async_copy(rows[b], acc_sh.at[si[b]], sems[b]).wait()

    def slot(j, b, finish, guard, prefetch):
        g2 = (b + NB - 2) % NB
        n = (b + 1) % NB
        wait_idx(b)
        start_gather(b)
        if finish:
            wait_gather(g2)
            scale(g2)
            start_scat(g2)
        if guard:
            wait_scat(n)
        if prefetch:
            start_idx(j + 1, n)

    start_idx(0, 0)
    slot(0, 0, False, False, True)
    slot(1, 1, False, False, True)
    slot(2, 2, True, False, True)

    def quad(u, _):
        j = 3 + 4 * u
        for k in range(4):
            slot(j + k, (3 + k) % NB, True, True, True)
        return 0

    lax.fori_loop(0, (NCH - 4) // 4, quad, 0)
    slot(NCH - 1, (NCH - 1) % NB, True, True, False)
    for c in (NCH - 2, NCH - 1):
        b = c % NB
        wait_gather(b)
        scale(b)
        start_scat(b)
    wait_scat((NCH - 3) % NB)
    wait_scat((NCH - 2) % NB)
    wait_scat((NCH - 1) % NB)
    plsc.subcore_barrier()

    for m in range(-(-NWB // NS)):
        jj = sid + NS * m

        @pl.when(jj < NWB)
        def _():
            pltpu.sync_copy(acc_sh.at[pl.ds(jj * WBR, WBR)], wb)
            pltpu.sync_copy(
                wb, out_hbm.at[pl.ds(cid * (PADF // H) + jj * WBR, WBR)])


_edge_v2c = _make_edge_pass(0, 1)
_edge_c2v = _make_edge_pass(1, 0)


N8 = NV // 8
PR = 12800
WB = 512
NG = PR // WB
FB = WB * 128
PADF = PR * 128


def _flat_spec(off=0):
    return pl.BlockSpec((FB,), lambda i, off=off: (i + off,))


def _full_spec(shape):
    return pl.BlockSpec(shape, lambda i: (0,) * len(shape))


def _prep_body(vc_ref, vx_ref, rv_ref, e8_ref, wv0_ref, wv1_ref, bv_ref,
               w2b_ref, out_ref):
    c = vc_ref[...] @ e8_ref[...]
    x = vx_ref[...] @ e8_ref[...]
    xv = jnp.maximum(c * wv0_ref[...] + x * wv1_ref[...] + bv_ref[...], 0.0)
    y = xv @ w2b_ref[...]
    out_ref[...] = (y * rv_ref[...].reshape(WB, 128)).reshape(FB)


def _prep(vc8, vx8, rv, e8, wv0, wv1, bv, w2b):
    return pl.pallas_call(
        _prep_body,
        grid=(NG,),
        in_specs=[pl.BlockSpec((WB, 8), lambda i: (i, 0)),
                  pl.BlockSpec((WB, 8), lambda i: (i, 0)),
                  _flat_spec(),
                  _full_spec((8, 128)), _full_spec((1, 128)),
                  _full_spec((1, 128)), _full_spec((1, 128)),
                  _full_spec((128, 128))],
        out_specs=_flat_spec(),
        out_shape=jax.ShapeDtypeStruct((PADF,), jnp.float32),
    )(vc8, vx8, rv, e8, wv0, wv1, bv, w2b)


def _mid_body(a0_ref, a1_ref, rc_ref, w2b_ref, b2_ref, out_ref):
    r = rc_ref[...].reshape(WB, 128)
    a = (a0_ref[...] + a1_ref[...]).reshape(WB, 128)
    h = jnp.maximum(a * r + b2_ref[...], 0.0)
    out_ref[...] = ((h @ w2b_ref[...]) * r).reshape(FB)


def _mid(agg, rc, w2b, b2t):
    return pl.pallas_call(
        _mid_body,
        grid=(NG,),
        in_specs=[_flat_spec(0), _flat_spec(NG), _flat_spec(),
                  _full_spec((128, 128)), _full_spec((1, 128))],
        out_specs=_flat_spec(),
        out_shape=jax.ShapeDtypeStruct((PADF,), jnp.float32),
    )(agg, agg, rc, w2b, b2t)


def _final_body(a0_ref, a1_ref, rv_ref, b2_ref, wo1_ref, bo1_ref,
                wo2_ref, bo2_ref, wo3_ref, bo3_ref, out_ref):
    @pl.when(pl.program_id(0) == 0)
    def _():
        out_ref[...] = bo3_ref[...]

    r = rv_ref[...].reshape(WB, 128)
    a = (a0_ref[...] + a1_ref[...]).reshape(WB, 128)
    h = jnp.maximum(a * r + b2_ref[...], 0.0)
    l = jnp.maximum(h @ wo1_ref[...] + bo1_ref[...], 0.0)
    l = jnp.maximum(l @ wo2_ref[...] + bo2_ref[...], 0.0)
    l3 = l @ wo3_ref[...]
    row = lax.broadcasted_iota(jnp.int32, (WB, 1), 0) + pl.program_id(0) * WB
    l3 = jnp.where(row < N8, l3, 0.0)
    out_ref[...] += jnp.sum(l3, keepdims=True)[:1, :1] * (1.0 / NV)


def _final(agg, rv, b2t, wo1b, bo1t, wo2b, bo2t, wo3b, bo3):
    return pl.pallas_call(
        _final_body,
        grid=(NG,),
        in_specs=[_flat_spec(0), _flat_spec(NG), _flat_spec(),
                  _full_spec((1, 128)),
                  _full_spec((128, 128)), _full_spec((1, 128)),
                  _full_spec((128, 128)), _full_spec((1, 128)),
                  _full_spec((128, 8)), _full_spec((1, 1))],
        out_specs=pl.BlockSpec((1, 1), lambda i: (0, 0)),
        out_shape=jax.ShapeDtypeStruct((1, 1), jnp.float32),
    )(agg, agg, rv, b2t, wo1b, bo1t, wo2b, bo2t, wo3b, bo3)


def kernel(var_c, var_x, con_b, edge_index, edge_A,
           W_ve, b_ve, W_ce, b_ce, W1, b1, W2, b2,
           Wo1, bo1, Wo2, bo2, Wo3, bo3):
    f32 = jnp.float32
    eye8 = jnp.eye(8, dtype=f32)
    e8 = jnp.kron(eye8, jnp.ones((1, H), f32))
    w2b = jnp.kron(eye8, W2)
    wo1b = jnp.kron(eye8, Wo1)
    wo2b = jnp.kron(eye8, Wo2)
    wo3b = jnp.kron(eye8, Wo3)
    wv0 = jnp.tile(W_ve[0], 8).reshape(1, 128)
    wv1 = jnp.tile(W_ve[1], 8).reshape(1, 128)
    bvt = jnp.tile(b_ve, 8).reshape(1, 128)
    b2t = jnp.tile(b2, 8).reshape(1, 128)
    bo1t = jnp.tile(bo1, 8).reshape(1, 128)
    bo2t = jnp.tile(bo2, 8).reshape(1, 128)

    rv, rc = _degrees(edge_index)

    pad8 = ((0, PR - N8), (0, 0))
    hs = _prep(jnp.pad(var_c.reshape(N8, 8), pad8),
               jnp.pad(var_x.reshape(N8, 8), pad8),
               rv, e8, wv0, wv1, bvt, w2b)
    agg_c = _edge_v2c(hs.reshape(PADF // H, H), edge_index, edge_A)
    gs = _mid(agg_c.reshape(NC * PADF), rc, w2b, b2t)
    agg_v = _edge_c2v(gs.reshape(PADF // H, H), edge_index, edge_A)
    return _final(agg_v.reshape(NC * PADF), rv, b2t, wo1b, bo1t, wo2b, bo2t,
                  wo3b, bo3.reshape(1, 1))


# speedup vs baseline: 1.2661x; 1.2661x over previous
import functools

import jax
import jax.numpy as jnp
from jax import lax
from jax.experimental import pallas as pl
from jax.experimental.pallas import tpu as pltpu
from jax.experimental.pallas import tpu_sc as plsc

NV = 100000
E = 3200000
H = 16

NC = 2
NS = 16
NW = NC * NS
EPW = E // NW
CH = 400
NCH = EPW // CH
NB = 3
EPT = E // NS
CHD = 2000
NCHD = EPT // CHD
NZCHD = NV // CHD

_mesh = plsc.VectorSubcoreMesh(core_axis_name="c", subcore_axis_name="s")


def _fill(ref, n, value):
    vec = jnp.full((16,), value, ref.dtype)

    @plsc.parallel_loop(0, n, 16)
    def _(i):
        ref[pl.ds(i, 16)] = vec


def _rsqrt16(x):
    i = lax.bitcast_convert_type(x, jnp.int32)
    i = 0x5F3759DF - lax.shift_right_logical(i, 1)
    y = lax.bitcast_convert_type(i, jnp.float32)
    for _ in range(3):
        y = y * (1.5 - 0.5 * x * y * y)
    return y


@functools.partial(
    pl.kernel,
    out_type=(
        jax.ShapeDtypeStruct((12800 * 128,), jnp.float32),
        jax.ShapeDtypeStruct((12800 * 128,), jnp.float32),
    ),
    mesh=_mesh,
    scratch_types=[
        pltpu.VMEM((CHD,), jnp.int32),
        pltpu.VMEM((CHD,), jnp.int32),
        pltpu.VMEM((CHD,), jnp.float32),
        pltpu.VMEM((CHD * H,), jnp.float32),
        pltpu.VMEM_SHARED((NV,), jnp.float32),
        pltpu.SemaphoreType.DMA,
        pltpu.SemaphoreType.DMA,
        pltpu.SemaphoreType.DMA,
        pltpu.SemaphoreType.DMA,
    ],
    compiler_params=pltpu.CompilerParams(use_tc_tiling_on_sc=False),
)
def _degrees(ei_hbm, rv_out, rc_out,
             sv0, sv1, ones_v, stage1d, hist_sh,
             semi0, semi1, sems0, sems1):
    cid = lax.axis_index("c")
    sid = lax.axis_index("s")
    sv = (sv0, sv1)
    semi = (semi0, semi1)
    sems = (sems0, sems1)

    _fill(ones_v, CHD, 0.0)
    for m in range(-(-NZCHD // NS)):
        jj = sid + NS * m

        @pl.when(jj < NZCHD)
        def _():
            pltpu.sync_copy(ones_v, hist_sh.at[pl.ds(jj * CHD, CHD)])
    _fill(ones_v, CHD, 1.0)
    plsc.subcore_barrier()

    def start_idx(j, b):
        base = sid * EPT + j * CHD
        pltpu.async_copy(ei_hbm.at[cid, pl.ds(base, CHD)], sv[b], semi[b])

    def wait_idx(b):
        pltpu.make_async_copy(ei_hbm.at[0, pl.ds(0, CHD)], sv[b], semi[b]).wait()

    def start_scat(b):
        pltpu.async_copy(ones_v, hist_sh.at[sv[b]], sems[b], add=True)

    def wait_scat(b):
        pltpu.make_async_copy(ones_v, hist_sh.at[sv[b]], sems[b]).wait()

    start_idx(0, 0)
    wait_idx(0)
    start_scat(0)
    start_idx(1, 1)

    def pair(t, _):
        for k in range(2):
            j = 2 * t + 1 + k
            b = 1 - k
            wait_idx(b)
            start_scat(b)
            wait_scat(1 - b)

            @pl.when(j < NCHD - 1)
            def _():
                start_idx(j + 1, 1 - b)
        return 0

    lax.fori_loop(0, (NCHD - 1) // 2, pair, 0)
    wait_idx(1)
    start_scat(1)
    wait_scat(0)
    wait_scat(1)
    plsc.subcore_barrier()

    for m in range(-(-NZCHD // NS)):
        jj = sid + NS * m

        @pl.when(jj < NZCHD)
        def _():
            pltpu.sync_copy(hist_sh.at[pl.ds(jj * CHD, CHD)], ones_v)

            @plsc.parallel_loop(0, CHD, 16)
            def _(i):
                d = jnp.maximum(ones_v[pl.ds(i, 16)], 1.0)
                y16 = _rsqrt16(d)
                for k in range(16):
                    stage1d[pl.ds((i + k) * H, H)] = jnp.full((H,), y16[k])

            @pl.when(cid == 0)
            def _():
                pltpu.sync_copy(stage1d, rv_out.at[pl.ds(jj * CHD * H, CHD * H)])

            @pl.when(cid == 1)
            def _():
                pltpu.sync_copy(stage1d, rc_out.at[pl.ds(jj * CHD * H, CHD * H)])


WBR = 400
NWB = NV // WBR


def _make_edge_pass(grow, srow):
    @functools.partial(
        pl.kernel,
        out_type=jax.ShapeDtypeStruct((NC * 12800 * 8, H), jnp.float32),
        mesh=_mesh,
        scratch_types=(
            [pltpu.VMEM((CH,), jnp.int32) for _ in range(NB)]
            + [pltpu.VMEM((CH,), jnp.int32) for _ in range(NB)]
            + [pltpu.VMEM((CH,), jnp.float32) for _ in range(NB)]
            + [pltpu.VMEM((CH, H), jnp.float32) for _ in range(NB)]
            + [pltpu.VMEM((WBR, H), jnp.float32)]
            + [pltpu.VMEM_SHARED((NV, H), jnp.float32)]
            + [pltpu.SemaphoreType.DMA for _ in range(3 * NB)]
        ),
        compiler_params=pltpu.CompilerParams(use_tc_tiling_on_sc=False),
    )
    def _edge_pass(table_hbm, ei_hbm, ew_hbm, out_hbm, *bufs):
        gi = bufs[0:NB]
        si = bufs[NB:2 * NB]
        ew = bufs[2 * NB:3 * NB]
        rows = bufs[3 * NB:4 * NB]
        wb = bufs[4 * NB]
        acc_sh = bufs[4 * NB + 1]
        semi = bufs[4 * NB + 2:4 * NB + 2 + NB]
        semg = bufs[4 * NB + 2 + NB:4 * NB + 2 + 2 * NB]
        sems = bufs[4 * NB + 2 + 2 * NB:4 * NB + 2 + 3 * NB]
        _edge_pass_body(grow, srow, table_hbm, ei_hbm, ew_hbm, out_hbm,
                        gi, si, ew, rows, wb, acc_sh, semi, semg, sems)

    return _edge_pass


def _edge_pass_body(grow, srow, table_hbm, ei_hbm, ew_hbm, out_hbm,
                    gi, si, ew, rows, wb, acc_sh, semi, semg, sems):
    cid = lax.axis_index("c")
    sid = lax.axis_index("s")
    wid = cid * NS + sid

    zvec = jnp.zeros((16,), jnp.float32)

    @plsc.parallel_loop(0, WBR, 1)
    def _(e):
        wb[e, :] = zvec

    for m in range(-(-NWB // NS)):
        jj = sid + NS * m

        @pl.when(jj < NWB)
        def _():
            pltpu.sync_copy(wb, acc_sh.at[pl.ds(jj * WBR, WBR)])
    plsc.subcore_barrier()

    def start_idx(j, b):
        base = wid * EPW + j * CH
        pltpu.async_copy(ei_hbm.at[grow, pl.ds(base, CH)], gi[b], semi[b])
        pltpu.async_copy(ei_hbm.at[srow, pl.ds(base, CH)], si[b], semi[b])
        pltpu.async_copy(ew_hbm.at[pl.ds(base, CH)], ew[b], semi[b])

    def wait_idx(b):
        pltpu.make_async_copy(ei_hbm.at[grow, pl.ds(0, CH)], gi[b], semi[b]).wait()
        pltpu.make_async_copy(ei_hbm.at[srow, pl.ds(0, CH)], si[b], semi[b]).wait()
        pltpu.make_async_copy(ew_hbm.at[pl.ds(0, CH)], ew[b], semi[b]).wait()

    def start_gather(b):
        pltpu.async_copy(table_hbm.at[gi[b]], rows[b], semg[b])

    def wait_gather(b):
        pltpu.make_async_copy(table_hbm.at[gi[b]], rows[b], semg[b]).wait()

    def scale(b):
        rv = rows[b]
        ev = ew[b]

        @plsc.parallel_loop(0, CH, 16)
        def _(e):
            w16 = ev[pl.ds(e, 16)]
            for k in range(16):
                rv[e + k, :] = rv[e + k, :] * w16[k]

    def start_scat(b):
        pltpu.async_copy(rows[b], acc_sh.at[si[b]], sems[b], add=True)

    def wait_scat(b):
        pltpu.make_async_copy(rows[b], acc_sh.at[si[b]], sems[b]).wait()

    def slot(j, b, finish, guard, prefetch):
        p = (b + NB - 1) % NB
        n = (b + 1) % NB
        wait_idx(b)
        start_gather(b)
        if finish:
            wait_gather(p)
            scale(p)
            start_scat(p)
        if guard:
            wait_scat(n)
        if prefetch:
            start_idx(j + 1, n)

    start_idx(0, 0)
    slot(0, 0, False, False, True)
    slot(1, 1, True, False, True)

    def tri(u, _):
        j = 2 + 3 * u
        for k in range(3):
            slot(j + k, (2 + k) % NB, True, True, True)
        return 0

    lax.fori_loop(0, (NCH - 4) // 3, tri, 0)
    slot(NCH - 2, (NCH - 2) % NB, True, True, True)
    slot(NCH - 1, (NCH - 1) % NB, True, True, False)
    b_last = (NCH - 1) % NB
    wait_gather(b_last)
    scale(b_last)
    start_scat(b_last)
    wait_scat((NCH - 2) % NB)
    wait_scat(b_last)
    plsc.subcore_barrier()

    for m in range(-(-NWB // NS)):
        jj = sid + NS * m

        @pl.when(jj < NWB)
        def _():
            pltpu.sync_copy(acc_sh.at[pl.ds(jj * WBR, WBR)], wb)
            pltpu.sync_copy(
                wb, out_hbm.at[pl.ds(cid * (PADF // H) + jj * WBR, WBR)])


_edge_v2c = _make_edge_pass(0, 1)
_edge_c2v = _make_edge_pass(1, 0)


N8 = NV // 8
PR = 12800
WB = 512
NG = PR // WB
FB = WB * 128
PADF = PR * 128


def _flat_spec(off=0):
    return pl.BlockSpec((FB,), lambda i, off=off: (i + off,))


def _full_spec(shape):
    return pl.BlockSpec(shape, lambda i: (0,) * len(shape))


def _prep_body(vc_ref, vx_ref, rv_ref, e8_ref, wv0_ref, wv1_ref, bv_ref,
               w2b_ref, out_ref):
    c = vc_ref[...] @ e8_ref[...]
    x = vx_ref[...] @ e8_ref[...]
    xv = jnp.maximum(c * wv0_ref[...] + x * wv1_ref[...] + bv_ref[...], 0.0)
    y = xv @ w2b_ref[...]
    out_ref[...] = (y * rv_ref[...].reshape(WB, 128)).reshape(FB)


def _prep(vc8, vx8, rv, e8, wv0, wv1, bv, w2b):
    return pl.pallas_call(
        _prep_body,
        grid=(NG,),
        in_specs=[pl.BlockSpec((WB, 8), lambda i: (i, 0)),
                  pl.BlockSpec((WB, 8), lambda i: (i, 0)),
                  _flat_spec(),
                  _full_spec((8, 128)), _full_spec((1, 128)),
                  _full_spec((1, 128)), _full_spec((1, 128)),
                  _full_spec((128, 128))],
        out_specs=_flat_spec(),
        out_shape=jax.ShapeDtypeStruct((PADF,), jnp.float32),
    )(vc8, vx8, rv, e8, wv0, wv1, bv, w2b)


def _mid_body(a0_ref, a1_ref, rc_ref, w2b_ref, b2_ref, out_ref):
    r = rc_ref[...].reshape(WB, 128)
    a = (a0_ref[...] + a1_ref[...]).reshape(WB, 128)
    h = jnp.maximum(a * r + b2_ref[...], 0.0)
    out_ref[...] = ((h @ w2b_ref[...]) * r).reshape(FB)


def _mid(agg, rc, w2b, b2t):
    return pl.pallas_call(
        _mid_body,
        grid=(NG,),
        in_specs=[_flat_spec(0), _flat_spec(NG), _flat_spec(),
                  _full_spec((128, 128)), _full_spec((1, 128))],
        out_specs=_flat_spec(),
        out_shape=jax.ShapeDtypeStruct((PADF,), jnp.float32),
    )(agg, agg, rc, w2b, b2t)


def _final_body(a0_ref, a1_ref, rv_ref, b2_ref, wo1_ref, bo1_ref,
                wo2_ref, bo2_ref, wo3_ref, bo3_ref, out_ref):
    @pl.when(pl.program_id(0) == 0)
    def _():
        out_ref[...] = bo3_ref[...]

    r = rv_ref[...].reshape(WB, 128)
    a = (a0_ref[...] + a1_ref[...]).reshape(WB, 128)
    h = jnp.maximum(a * r + b2_ref[...], 0.0)
    l = jnp.maximum(h @ wo1_ref[...] + bo1_ref[...], 0.0)
    l = jnp.maximum(l @ wo2_ref[...] + bo2_ref[...], 0.0)
    l3 = l @ wo3_ref[...]
    row = lax.broadcasted_iota(jnp.int32, (WB, 1), 0) + pl.program_id(0) * WB
    l3 = jnp.where(row < N8, l3, 0.0)
    out_ref[...] += jnp.sum(l3, keepdims=True)[:1, :1] * (1.0 / NV)


def _final(agg, rv, b2t, wo1b, bo1t, wo2b, bo2t, wo3b, bo3):
    return pl.pallas_call(
        _final_body,
        grid=(NG,),
        in_specs=[_flat_spec(0), _flat_spec(NG), _flat_spec(),
                  _full_spec((1, 128)),
                  _full_spec((128, 128)), _full_spec((1, 128)),
                  _full_spec((128, 128)), _full_spec((1, 128)),
                  _full_spec((128, 8)), _full_spec((1, 1))],
        out_specs=pl.BlockSpec((1, 1), lambda i: (0, 0)),
        out_shape=jax.ShapeDtypeStruct((1, 1), jnp.float32),
    )(agg, agg, rv, b2t, wo1b, bo1t, wo2b, bo2t, wo3b, bo3)


def kernel(var_c, var_x, con_b, edge_index, edge_A,
           W_ve, b_ve, W_ce, b_ce, W1, b1, W2, b2,
           Wo1, bo1, Wo2, bo2, Wo3, bo3):
    f32 = jnp.float32
    eye8 = jnp.eye(8, dtype=f32)
    e8 = jnp.kron(eye8, jnp.ones((1, H), f32))
    w2b = jnp.kron(eye8, W2)
    wo1b = jnp.kron(eye8, Wo1)
    wo2b = jnp.kron(eye8, Wo2)
    wo3b = jnp.kron(eye8, Wo3)
    wv0 = jnp.tile(W_ve[0], 8).reshape(1, 128)
    wv1 = jnp.tile(W_ve[1], 8).reshape(1, 128)
    bvt = jnp.tile(b_ve, 8).reshape(1, 128)
    b2t = jnp.tile(b2, 8).reshape(1, 128)
    bo1t = jnp.tile(bo1, 8).reshape(1, 128)
    bo2t = jnp.tile(bo2, 8).reshape(1, 128)

    rv, rc = _degrees(edge_index)

    pad8 = ((0, PR - N8), (0, 0))
    hs = _prep(jnp.pad(var_c.reshape(N8, 8), pad8),
               jnp.pad(var_x.reshape(N8, 8), pad8),
               rv, e8, wv0, wv1, bvt, w2b)
    agg_c = _edge_v2c(hs.reshape(PADF // H, H), edge_index, edge_A)
    gs = _mid(agg_c.reshape(NC * PADF), rc, w2b, b2t)
    agg_v = _edge_c2v(gs.reshape(PADF // H, H), edge_index, edge_A)
    return _final(agg_v.reshape(NC * PADF), rv, b2t, wo1b, bo1t, wo2b, bo2t,
                  wo3b, bo3.reshape(1, 1))


# speedup vs baseline: 1.2835x; 1.0138x over previous
import functools

import jax
import jax.numpy as jnp
from jax import lax
from jax.experimental import pallas as pl
from jax.experimental.pallas import tpu as pltpu
from jax.experimental.pallas import tpu_sc as plsc

NV = 100000
E = 3200000
H = 16

NC = 2
NS = 16
NW = NC * NS
EPW = E // NW
CH = 400
NCH = EPW // CH
NB = 3
EPT = E // NS
CHD = 2000
NCHD = EPT // CHD
NZCHD = NV // CHD

_mesh = plsc.VectorSubcoreMesh(core_axis_name="c", subcore_axis_name="s")


def _fill(ref, n, value):
    vec = jnp.full((16,), value, ref.dtype)

    @plsc.parallel_loop(0, n, 16)
    def _(i):
        ref[pl.ds(i, 16)] = vec


def _rsqrt16(x):
    i = lax.bitcast_convert_type(x, jnp.int32)
    i = 0x5F3759DF - lax.shift_right_logical(i, 1)
    y = lax.bitcast_convert_type(i, jnp.float32)
    for _ in range(3):
        y = y * (1.5 - 0.5 * x * y * y)
    return y


@functools.partial(
    pl.kernel,
    out_type=(
        jax.ShapeDtypeStruct((12800 * 128,), jnp.float32),
        jax.ShapeDtypeStruct((12800 * 128,), jnp.float32),
    ),
    mesh=_mesh,
    scratch_types=[
        pltpu.VMEM((CHD,), jnp.int32),
        pltpu.VMEM((CHD,), jnp.int32),
        pltpu.VMEM((CHD,), jnp.float32),
        pltpu.VMEM((CHD * H,), jnp.float32),
        pltpu.VMEM_SHARED((NV,), jnp.float32),
        pltpu.SemaphoreType.DMA,
        pltpu.SemaphoreType.DMA,
        pltpu.SemaphoreType.DMA,
        pltpu.SemaphoreType.DMA,
    ],
    compiler_params=pltpu.CompilerParams(use_tc_tiling_on_sc=False),
)
def _degrees(ei_hbm, rv_out, rc_out,
             sv0, sv1, ones_v, stage1d, hist_sh,
             semi0, semi1, sems0, sems1):
    cid = lax.axis_index("c")
    sid = lax.axis_index("s")
    sv = (sv0, sv1)
    semi = (semi0, semi1)
    sems = (sems0, sems1)

    _fill(ones_v, CHD, 0.0)
    for m in range(-(-NZCHD // NS)):
        jj = sid + NS * m

        @pl.when(jj < NZCHD)
        def _():
            pltpu.sync_copy(ones_v, hist_sh.at[pl.ds(jj * CHD, CHD)])
    _fill(ones_v, CHD, 1.0)
    plsc.subcore_barrier()

    def start_idx(j, b):
        base = sid * EPT + j * CHD
        pltpu.async_copy(ei_hbm.at[cid, pl.ds(base, CHD)], sv[b], semi[b])

    def wait_idx(b):
        pltpu.make_async_copy(ei_hbm.at[0, pl.ds(0, CHD)], sv[b], semi[b]).wait()

    def start_scat(b):
        pltpu.async_copy(ones_v, hist_sh.at[sv[b]], sems[b], add=True)

    def wait_scat(b):
        pltpu.make_async_copy(ones_v, hist_sh.at[sv[b]], sems[b]).wait()

    start_idx(0, 0)
    wait_idx(0)
    start_scat(0)
    start_idx(1, 1)

    def pair(t, _):
        for k in range(2):
            j = 2 * t + 1 + k
            b = 1 - k
            wait_idx(b)
            start_scat(b)
            wait_scat(1 - b)

            @pl.when(j < NCHD - 1)
            def _():
                start_idx(j + 1, 1 - b)
        return 0

    lax.fori_loop(0, (NCHD - 1) // 2, pair, 0)
    wait_idx(1)
    start_scat(1)
    wait_scat(0)
    wait_scat(1)
    plsc.subcore_barrier()

    for m in range(-(-NZCHD // NS)):
        jj = sid + NS * m

        @pl.when(jj < NZCHD)
        def _():
            pltpu.sync_copy(hist_sh.at[pl.ds(jj * CHD, CHD)], ones_v)

            @plsc.parallel_loop(0, CHD, 16)
            def _(i):
                d = jnp.maximum(ones_v[pl.ds(i, 16)], 1.0)
                y16 = _rsqrt16(d)
                for k in range(16):
                    stage1d[pl.ds((i + k) * H, H)] = jnp.full((H,), y16[k])

            @pl.when(cid == 0)
            def _():
                pltpu.sync_copy(stage1d, rv_out.at[pl.ds(jj * CHD * H, CHD * H)])

            @pl.when(cid == 1)
            def _():
                pltpu.sync_copy(stage1d, rc_out.at[pl.ds(jj * CHD * H, CHD * H)])


WBR = 400
NWB = NV // WBR


def _make_edge_pass(grow, srow):
    @functools.partial(
        pl.kernel,
        out_type=jax.ShapeDtypeStruct((NC * 12800 * 8, H), jnp.float32),
        mesh=_mesh,
        scratch_types=(
            [pltpu.VMEM((CH,), jnp.int32) for _ in range(NB)]
            + [pltpu.VMEM((CH,), jnp.int32) for _ in range(NB)]
            + [pltpu.VMEM((CH,), jnp.float32) for _ in range(NB)]
            + [pltpu.VMEM((CH, H), jnp.float32) for _ in range(NB)]
            + [pltpu.VMEM((WBR, H), jnp.float32)]
            + [pltpu.VMEM_SHARED((NV, H), jnp.float32)]
            + [pltpu.SemaphoreType.DMA for _ in range(3 * NB)]
        ),
        compiler_params=pltpu.CompilerParams(use_tc_tiling_on_sc=False),
    )
    def _edge_pass(table_hbm, ei_hbm, ew_hbm, out_hbm, *bufs):
        gi = bufs[0:NB]
        si = bufs[NB:2 * NB]
        ew = bufs[2 * NB:3 * NB]
        rows = bufs[3 * NB:4 * NB]
        wb = bufs[4 * NB]
        acc_sh = bufs[4 * NB + 1]
        semi = bufs[4 * NB + 2:4 * NB + 2 + NB]
        semg = bufs[4 * NB + 2 + NB:4 * NB + 2 + 2 * NB]
        sems = bufs[4 * NB + 2 + 2 * NB:4 * NB + 2 + 3 * NB]
        _edge_pass_body(grow, srow, table_hbm, ei_hbm, ew_hbm, out_hbm,
                        gi, si, ew, rows, wb, acc_sh, semi, semg, sems)

    return _edge_pass


def _edge_pass_body(grow, srow, table_hbm, ei_hbm, ew_hbm, out_hbm,
                    gi, si, ew, rows, wb, acc_sh, semi, semg, sems):
    cid = lax.axis_index("c")
    sid = lax.axis_index("s")
    wid = cid * NS + sid

    zvec = jnp.zeros((16,), jnp.float32)

    @plsc.parallel_loop(0, WBR, 1)
    def _(e):
        wb[e, :] = zvec

    nwbm = -(-NWB // NS)
    for m in range(nwbm):
        jj = sid + NS * m

        @pl.when(jj < NWB)
        def _():
            pltpu.async_copy(wb, acc_sh.at[pl.ds(jj * WBR, WBR)], semg[0])
    for m in range(nwbm):
        jj = sid + NS * m

        @pl.when(jj < NWB)
        def _():
            pltpu.make_async_copy(
                wb, acc_sh.at[pl.ds(jj * WBR, WBR)], semg[0]).wait()
    plsc.subcore_barrier()

    def start_idx(j, b):
        base = wid * EPW + j * CH
        pltpu.async_copy(ei_hbm.at[grow, pl.ds(base, CH)], gi[b], semi[b])
        pltpu.async_copy(ei_hbm.at[srow, pl.ds(base, CH)], si[b], semi[b])
        pltpu.async_copy(ew_hbm.at[pl.ds(base, CH)], ew[b], semi[b])

    def wait_idx(b):
        pltpu.make_async_copy(ei_hbm.at[grow, pl.ds(0, CH)], gi[b], semi[b]).wait()
        pltpu.make_async_copy(ei_hbm.at[srow, pl.ds(0, CH)], si[b], semi[b]).wait()
        pltpu.make_async_copy(ew_hbm.at[pl.ds(0, CH)], ew[b], semi[b]).wait()

    def start_gather(b):
        pltpu.async_copy(table_hbm.at[gi[b]], rows[b], semg[b])

    def wait_gather(b):
        pltpu.make_async_copy(table_hbm.at[gi[b]], rows[b], semg[b]).wait()

    def scale(b):
        rv = rows[b]
        ev = ew[b]

        @plsc.parallel_loop(0, CH, 16)
        def _(e):
            w16 = ev[pl.ds(e, 16)]
            for k in range(16):
                rv[e + k, :] = rv[e + k, :] * w16[k]

    def start_scat(b):
        pltpu.async_copy(rows[b], acc_sh.at[si[b]], sems[b], add=True)

    def wait_scat(b):
        pltpu.make_async_copy(rows[b], acc_sh.at[si[b]], sems[b]).wait()

    def slot(j, b, finish, guard, prefetch):
        p = (b + NB - 1) % NB
        n = (b + 1) % NB
        wait_idx(b)
        start_gather(b)
        if finish:
            wait_gather(p)
            scale(p)
            start_scat(p)
        if guard:
            wait_scat(n)
        if prefetch:
            start_idx(j + 1, n)

    start_idx(0, 0)
    slot(0, 0, False, False, True)
    slot(1, 1, True, False, True)

    def tri(u, _):
        j = 2 + 3 * u
        for k in range(3):
            slot(j + k, (2 + k) % NB, True, True, True)
        return 0

    lax.fori_loop(0, (NCH - 4) // 3, tri, 0)
    slot(NCH - 2, (NCH - 2) % NB, True, True, True)
    slot(NCH - 1, (NCH - 1) % NB, True, True, False)
    b_last = (NCH - 1) % NB
    wait_gather(b_last)
    scale(b_last)
    start_scat(b_last)
    wait_scat((NCH - 2) % NB)
    wait_scat(b_last)
    plsc.subcore_barrier()

    def _wb_out(jj, b):
        return out_hbm.at[pl.ds(cid * (PADF // H) + jj * WBR, WBR)]

    nwbm = -(-NWB // NS)
    for m in range(nwbm):
        if m >= 2:
            jp = sid + NS * (m - 2)
            bp = (m - 2) % 2

            @pl.when(jp < NWB)
            def _():
                pltpu.make_async_copy(rows[bp], _wb_out(jp, bp),
                                      semg[bp]).wait()
        jj = sid + NS * m
        b = m % 2

        @pl.when(jj < NWB)
        def _():
            pltpu.sync_copy(acc_sh.at[pl.ds(jj * WBR, WBR)], rows[b])
            pltpu.async_copy(rows[b], _wb_out(jj, b), semg[b])
    for m in (nwbm - 2, nwbm - 1):
        jp = sid + NS * m
        bp = m % 2

        @pl.when(jp < NWB)
        def _():
            pltpu.make_async_copy(rows[bp], _wb_out(jp, bp), semg[bp]).wait()


_edge_v2c = _make_edge_pass(0, 1)
_edge_c2v = _make_edge_pass(1, 0)


N8 = NV // 8
PR = 12800
WB = 512
NG = PR // WB
FB = WB * 128
PADF = PR * 128


def _flat_spec(off=0):
    return pl.BlockSpec((FB,), lambda i, off=off: (i + off,))


def _full_spec(shape):
    return pl.BlockSpec(shape, lambda i: (0,) * len(shape))


def _prep_body(vc_ref, vx_ref, rv_ref, e8_ref, wv0_ref, wv1_ref, bv_ref,
               w2b_ref, out_ref):
    c = vc_ref[...] @ e8_ref[...]
    x = vx_ref[...] @ e8_ref[...]
    xv = jnp.maximum(c * wv0_ref[...] + x * wv1_ref[...] + bv_ref[...], 0.0)
    y = xv @ w2b_ref[...]
    out_ref[...] = (y * rv_ref[...].reshape(WB, 128)).reshape(FB)


def _prep(vc8, vx8, rv, e8, wv0, wv1, bv, w2b):
    return pl.pallas_call(
        _prep_body,
        grid=(NG,),
        in_specs=[pl.BlockSpec((WB, 8), lambda i: (i, 0)),
                  pl.BlockSpec((WB, 8), lambda i: (i, 0)),
                  _flat_spec(),
                  _full_spec((8, 128)), _full_spec((1, 128)),
                  _full_spec((1, 128)), _full_spec((1, 128)),
                  _full_spec((128, 128))],
        out_specs=_flat_spec(),
        out_shape=jax.ShapeDtypeStruct((PADF,), jnp.float32),
    )(vc8, vx8, rv, e8, wv0, wv1, bv, w2b)


def _mid_body(a0_ref, a1_ref, rc_ref, w2b_ref, b2_ref, out_ref):
    r = rc_ref[...].reshape(WB, 128)
    a = (a0_ref[...] + a1_ref[...]).reshape(WB, 128)
    h = jnp.maximum(a * r + b2_ref[...], 0.0)
    out_ref[...] = ((h @ w2b_ref[...]) * r).reshape(FB)


def _mid(agg, rc, w2b, b2t):
    return pl.pallas_call(
        _mid_body,
        grid=(NG,),
        in_specs=[_flat_spec(0), _flat_spec(NG), _flat_spec(),
                  _full_spec((128, 128)), _full_spec((1, 128))],
        out_specs=_flat_spec(),
        out_shape=jax.ShapeDtypeStruct((PADF,), jnp.float32),
    )(agg, agg, rc, w2b, b2t)


def _final_body(a0_ref, a1_ref, rv_ref, b2_ref, wo1_ref, bo1_ref,
                wo2_ref, bo2_ref, wo3_ref, bo3_ref, out_ref):
    @pl.when(pl.program_id(0) == 0)
    def _():
        out_ref[...] = bo3_ref[...]

    r = rv_ref[...].reshape(WB, 128)
    a = (a0_ref[...] + a1_ref[...]).reshape(WB, 128)
    h = jnp.maximum(a * r + b2_ref[...], 0.0)
    l = jnp.maximum(h @ wo1_ref[...] + bo1_ref[...], 0.0)
    l = jnp.maximum(l @ wo2_ref[...] + bo2_ref[...], 0.0)
    l3 = l @ wo3_ref[...]
    row = lax.broadcasted_iota(jnp.int32, (WB, 1), 0) + pl.program_id(0) * WB
    l3 = jnp.where(row < N8, l3, 0.0)
    out_ref[...] += jnp.sum(l3, keepdims=True)[:1, :1] * (1.0 / NV)


def _final(agg, rv, b2t, wo1b, bo1t, wo2b, bo2t, wo3b, bo3):
    return pl.pallas_call(
        _final_body,
        grid=(NG,),
        in_specs=[_flat_spec(0), _flat_spec(NG), _flat_spec(),
                  _full_spec((1, 128)),
                  _full_spec((128, 128)), _full_spec((1, 128)),
                  _full_spec((128, 128)), _full_spec((1, 128)),
                  _full_spec((128, 8)), _full_spec((1, 1))],
        out_specs=pl.BlockSpec((1, 1), lambda i: (0, 0)),
        out_shape=jax.ShapeDtypeStruct((1, 1), jnp.float32),
    )(agg, agg, rv, b2t, wo1b, bo1t, wo2b, bo2t, wo3b, bo3)


def kernel(var_c, var_x, con_b, edge_index, edge_A,
           W_ve, b_ve, W_ce, b_ce, W1, b1, W2, b2,
           Wo1, bo1, Wo2, bo2, Wo3, bo3):
    f32 = jnp.float32
    eye8 = jnp.eye(8, dtype=f32)
    e8 = jnp.kron(eye8, jnp.ones((1, H), f32))
    w2b = jnp.kron(eye8, W2)
    wo1b = jnp.kron(eye8, Wo1)
    wo2b = jnp.kron(eye8, Wo2)
    wo3b = jnp.kron(eye8, Wo3)
    wv0 = jnp.tile(W_ve[0], 8).reshape(1, 128)
    wv1 = jnp.tile(W_ve[1], 8).reshape(1, 128)
    bvt = jnp.tile(b_ve, 8).reshape(1, 128)
    b2t = jnp.tile(b2, 8).reshape(1, 128)
    bo1t = jnp.tile(bo1, 8).reshape(1, 128)
    bo2t = jnp.tile(bo2, 8).reshape(1, 128)

    rv, rc = _degrees(edge_index)

    pad8 = ((0, PR - N8), (0, 0))
    hs = _prep(jnp.pad(var_c.reshape(N8, 8), pad8),
               jnp.pad(var_x.reshape(N8, 8), pad8),
               rv, e8, wv0, wv1, bvt, w2b)
    agg_c = _edge_v2c(hs.reshape(PADF // H, H), edge_index, edge_A)
    gs = _mid(agg_c.reshape(NC * PADF), rc, w2b, b2t)
    agg_v = _edge_c2v(gs.reshape(PADF // H, H), edge_index, edge_A)
    return _final(agg_v.reshape(NC * PADF), rv, b2t, wo1b, bo1t, wo2b, bo2t,
                  wo3b, bo3.reshape(1, 1))
